# Initial kernel scaffold; baseline (speedup 1.0000x reference)
import jax
import jax.numpy as jnp
from jax.experimental import pallas as pl


def kernel(logits, targets):
    raise NotImplementedError("write your pallas kernel here")


# speedup vs baseline: 19.2737x; 19.2737x over previous
import functools

import jax
import jax.numpy as jnp
from jax import lax
from jax.experimental import pallas as pl
from jax.experimental.pallas import tpu as pltpu
from jax.experimental.pallas import tpu_sc as plsc

_N_TOK = 32768
_ENT_W = 0.001
_EPS = 0.0001
_A_CLIP = 9.210240366975849
_H_CLIP = 0.0010210290545737

_NC = 2
_NS = 16
_NW = _NC * _NS
_L = 16
_CHUNK = _N_TOK // _NW
_NVEC = _CHUNK // _L


def _entropy_vec(x):
    a = jnp.abs(x)
    z = jnp.exp(-a)
    u = z / (2.0 + z)
    u2 = u * u
    log1pz = 2.0 * u * (
        1.0 + u2 * (1.0 / 3.0 + u2 * (1.0 / 5.0 + u2 * (1.0 / 7.0 + u2 * (1.0 / 9.0))))
    )
    h = log1pz + a * z / (1.0 + z)
    return jnp.where(a > _A_CLIP, _H_CLIP, h)


@functools.partial(
    pl.kernel,
    out_type=jax.ShapeDtypeStruct((_NC, _L), jnp.float32),
    mesh=plsc.VectorSubcoreMesh(core_axis_name="c", subcore_axis_name="s"),
    compiler_params=pltpu.CompilerParams(needs_layout_passes=False),
    scratch_types=[
        pltpu.VMEM((_CHUNK,), jnp.float32),
        pltpu.VMEM((_L,), jnp.float32),
        pltpu.VMEM((_NS * _L,), jnp.float32),
        pltpu.VMEM_SHARED((_NS * _L,), jnp.float32),
    ],
)
def _entropy_loss_kernel(x_hbm, out_hbm, xv, stage_v, all_v, shared):
    cid = lax.axis_index("c")
    sid = lax.axis_index("s")
    wid = cid * _NS + sid

    pltpu.sync_copy(x_hbm.at[pl.ds(wid * _CHUNK, _CHUNK)], xv)

    def body(i, acc):
        return acc + _entropy_vec(xv[pl.ds(i * _L, _L)])

    acc = lax.fori_loop(0, _NVEC, body, jnp.zeros((_L,), jnp.float32))

    stage_v[...] = acc
    pltpu.sync_copy(stage_v, shared.at[pl.ds(sid * _L, _L)])
    plsc.subcore_barrier()

    @pl.when(sid == 0)
    def _():
        pltpu.sync_copy(shared, all_v)

        def body2(j, tot):
            return tot + all_v[pl.ds(j * _L, _L)]

        tot = lax.fori_loop(0, _NS, body2, jnp.zeros((_L,), jnp.float32))
        core_partial = jnp.sum(tot) * (-_ENT_W / _N_TOK)
        stage_v[...] = jnp.full((_L,), core_partial, jnp.float32)
        pltpu.sync_copy(stage_v, out_hbm.at[cid])


def kernel(logits, targets):
    del targets
    out = _entropy_loss_kernel(logits.reshape(_N_TOK))
    return out[0, 0] + out[1, 0]
